# baseline (speedup 1.0000x reference)
import functools

import jax
import jax.numpy as jnp
from jax import lax
from jax.experimental import pallas as pl
from jax.experimental.pallas import tpu as pltpu
from jax.experimental.pallas import tpu_sc as plsc

N = 10000
E = 160000
D = 256
H = 128
MB = 400
NMB = N // MB
NS = 16
EPT = E // NS
CH = 128
NCHUNK = 80
EPTP = NCHUNK * CH
GSZ = 8
NGRP = NCHUNK // GSZ
NPAD = N + 8
CP_TILES = 10
CP_ROWS = N // CP_TILES


def _mesh():
    return plsc.VectorSubcoreMesh(
        core_axis_name="c", subcore_axis_name="s",
        num_cores=2, num_subcores=NS)


@functools.lru_cache(maxsize=None)
def _make_sc_segsum():
    return pl.kernel(
        _sc_segsum_body,
        out_type=(jax.ShapeDtypeStruct((2 * N, H), jnp.float32),
                  jax.ShapeDtypeStruct((2 * N, H), jnp.float32)),
        mesh=_mesh(),
        scratch_types=[
            pltpu.VMEM_SHARED((NPAD, H), jnp.float32),
            pltpu.VMEM((GSZ, CH), jnp.int32),
            [pltpu.VMEM((CH,), jnp.int32) for _ in range(GSZ)],
            pltpu.VMEM((CH, H), jnp.float32),
            pltpu.VMEM((CH, H), jnp.float32),
            pltpu.SemaphoreType.DMA,
            pltpu.SemaphoreType.DMA,
            pltpu.SemaphoreType.DMA,
        ],
    )


def _sc_segsum(tp, ta, row_g, col_g, row_s, col_s, zeros):
    return _make_sc_segsum()(tp, ta, row_g, col_g, row_s, col_s, zeros)


def _one_direction(c, s, w, table_ref, sidx_ref, didx_ref, zeros_ref,
                   out_ref, acc, idx2b, drefs, buf0, buf1, gsem, ssem, isem):
    @pl.when(s < CP_TILES)
    def _():
        pltpu.sync_copy(zeros_ref, acc.at[pl.ds(s * CP_ROWS, CP_ROWS)])

    plsc.subcore_barrier()
    gidx_v = drefs[0]
    didx_v = drefs[1]

    def body(j, carry):
        pltpu.sync_copy(sidx_ref.at[w, j], gidx_v)
        pltpu.sync_copy(didx_ref.at[s, j], didx_v)
        pltpu.async_copy(table_ref.at[gidx_v], buf0, gsem).wait()
        pltpu.sync_copy(buf0, acc.at[didx_v], add=True)
        return carry

    lax.fori_loop(0, NCHUNK, body, 0)
    plsc.subcore_barrier()

    @pl.when(s < CP_TILES)
    def _():
        row0 = c * N + s * CP_ROWS
        pltpu.sync_copy(acc.at[pl.ds(s * CP_ROWS, CP_ROWS)],
                        out_ref.at[pl.ds(row0, CP_ROWS)])


def _sc_segsum_body(tp_ref, ta_ref, row_g_ref, col_g_ref, row_s_ref,
                    col_s_ref, zeros_ref, outp_ref, outa_ref,
                    acc, idx2b, drefs, buf0, buf1, gsem, ssem, isem):
    c = lax.axis_index("c")
    s = lax.axis_index("s")
    w = c * NS + s
    _one_direction(c, s, w, tp_ref, row_g_ref, col_s_ref, zeros_ref,
                   outp_ref, acc, idx2b, drefs, buf0, buf1, gsem, ssem, isem)
    plsc.subcore_barrier()
    _one_direction(c, s, w, ta_ref, col_g_ref, row_s_ref, zeros_ref,
                   outa_ref, acc, idx2b, drefs, buf0, buf1, gsem, ssem, isem)


@functools.lru_cache(maxsize=None)
def _make_sc_counts():
    return pl.kernel(
        _sc_counts_body,
        out_type=jax.ShapeDtypeStruct((2 * N, H), jnp.float32),
        mesh=_mesh(),
        scratch_types=[
            pltpu.VMEM_SHARED((NPAD, H), jnp.float32),
            pltpu.VMEM((CH,), jnp.int32),
            pltpu.VMEM((CH, H), jnp.float32),
        ],
    )


def _sc_counts(didx2, zeros, ones):
    return _make_sc_counts()(didx2, zeros, ones)


def _sc_counts_body(didx_ref, zeros_ref, ones_ref, out_ref, cacc, didx_vm,
                    ones_v):
    c = lax.axis_index("c")
    s = lax.axis_index("s")

    @pl.when(s < CP_TILES)
    def _():
        pltpu.sync_copy(zeros_ref, cacc.at[pl.ds(s * CP_ROWS, CP_ROWS)])

    pltpu.sync_copy(ones_ref, ones_v)
    plsc.subcore_barrier()

    def body(i, carry):
        pltpu.sync_copy(didx_ref.at[c * NS + s, i], didx_vm)
        pltpu.sync_copy(ones_v, cacc.at[didx_vm], add=True)
        return carry

    lax.fori_loop(0, NCHUNK, body, 0)
    plsc.subcore_barrier()

    @pl.when(s < CP_TILES)
    def _():
        row0 = c * N + s * CP_ROWS
        pltpu.sync_copy(cacc.at[pl.ds(s * CP_ROWS, CP_ROWS)],
                        out_ref.at[pl.ds(row0, CP_ROWS)])


def _proj_body(x_ref, w_ref, b_ref, o_ref):
    h = jnp.dot(x_ref[...], w_ref[...], preferred_element_type=jnp.float32)
    o_ref[...] = jnp.maximum(h + b_ref[...], 0.0)


def _tc_proj(x, w, b):
    return pl.pallas_call(
        _proj_body,
        grid=(NMB,),
        in_specs=[
            pl.BlockSpec((MB, D), lambda m: (m, 0)),
            pl.BlockSpec((D, D), lambda m: (0, 0)),
            pl.BlockSpec((1, D), lambda m: (0, 0)),
        ],
        out_specs=pl.BlockSpec((MB, D), lambda m: (m, 0)),
        out_shape=jax.ShapeDtypeStruct((N, D), jnp.float32),
    )(x, w, b)


def _post_body(norm_relu, acc_ref, cnt_ref, xd_ref, wl_ref, bl_ref, wr_ref,
               o_ref):
    denom = jnp.maximum(cnt_ref[...], 1.0)
    a0 = acc_ref[0] / denom
    a1 = acc_ref[1] / denom
    out = (jnp.dot(a0, wl_ref[0], preferred_element_type=jnp.float32)
           + jnp.dot(a1, wl_ref[1], preferred_element_type=jnp.float32)
           + jnp.dot(xd_ref[...], wr_ref[...],
                     preferred_element_type=jnp.float32)
           + bl_ref[...])
    if norm_relu:
        n = jnp.sqrt(jnp.sum(out * out, axis=1, keepdims=True))
        out = out / jnp.maximum(n, 1e-12)
        out = jnp.maximum(out, 0.0)
    o_ref[...] = out


def _tc_post(acc, cnt, xd, wl, bl, wr, norm_relu):
    return pl.pallas_call(
        functools.partial(_post_body, norm_relu),
        grid=(NMB,),
        in_specs=[
            pl.BlockSpec((2, MB, H), lambda m: (0, m, 0)),
            pl.BlockSpec((MB, 1), lambda m: (m, 0)),
            pl.BlockSpec((MB, D), lambda m: (m, 0)),
            pl.BlockSpec((2, H, D), lambda m: (0, 0, 0)),
            pl.BlockSpec((1, D), lambda m: (0, 0)),
            pl.BlockSpec((D, D), lambda m: (0, 0)),
        ],
        out_specs=pl.BlockSpec((MB, D), lambda m: (m, 0)),
        out_shape=jax.ShapeDtypeStruct((N, D), jnp.float32),
    )(acc, cnt, xd, wl, bl, wr)


def kernel(x_author, x_paper, edge_index, W_proj, b_proj, W_l, b_l, W_r):
    row2 = edge_index[0].reshape(NS, EPT)
    col2 = edge_index[1].reshape(NS, EPT)
    pad = EPTP - EPT
    row_gp = jnp.pad(row2, ((0, 0), (0, pad)))
    col_gp = jnp.pad(col2, ((0, 0), (0, pad)))
    row_g = jnp.stack([row_gp * 2, row_gp * 2 + 1]).reshape(
        2 * NS, NCHUNK, CH)
    col_g = jnp.stack([col_gp * 2, col_gp * 2 + 1]).reshape(
        2 * NS, NCHUNK, CH)
    row_s = jnp.pad(row2, ((0, 0), (0, pad)),
                    constant_values=N).reshape(NS, NCHUNK, CH)
    col_s = jnp.pad(col2, ((0, 0), (0, pad)),
                    constant_values=N).reshape(NS, NCHUNK, CH)
    zeros_h = jnp.zeros((CP_ROWS, H), jnp.float32)
    ones_h = jnp.ones((CH, H), jnp.float32)

    didx2 = jnp.concatenate([col_s, row_s], axis=0)
    counts = _sc_counts(didx2, zeros_h, ones_h)
    cnt_p = counts[0:N, 0:1]
    cnt_a = counts[N:2 * N, 0:1]

    def seg_both(ta_for_p, tp_for_a):
        op, oa = _sc_segsum(ta_for_p.reshape(2 * N, H),
                            tp_for_a.reshape(2 * N, H),
                            row_g, col_g, row_s, col_s, zeros_h)
        return op.reshape(2, N, H), oa.reshape(2, N, H)

    xa, xp = x_author, x_paper
    for i in range(4):
        ha = _tc_proj(xa, W_proj[i, 0], b_proj[i, 0].reshape(1, D))
        hp = _tc_proj(xp, W_proj[i, 1], b_proj[i, 1].reshape(1, D))
        sp, sa = seg_both(ha, hp)
        xp_new = _tc_post(sp, cnt_p, xp, W_l[i, 0].reshape(2, H, D),
                          b_l[i, 0].reshape(1, D), W_r[i, 0], True)
        xa_new = _tc_post(sa, cnt_a, xa, W_l[i, 1].reshape(2, H, D),
                          b_l[i, 1].reshape(1, D), W_r[i, 1], True)
        xp, xa = xp_new, xa_new

    sp, sa = seg_both(xa, xp)
    out_p = _tc_post(sp, cnt_p, xp, W_l[4, 0].reshape(2, H, D),
                     b_l[4, 0].reshape(1, D), W_r[4, 0], False)
    out_a = _tc_post(sa, cnt_a, xa, W_l[4, 1].reshape(2, H, D),
                     b_l[4, 1].reshape(1, D), W_r[4, 1], False)
    return (out_a, out_p)


# speedup vs baseline: 1.3818x; 1.3818x over previous
import functools

import jax
import jax.numpy as jnp
from jax import lax
from jax.experimental import pallas as pl
from jax.experimental.pallas import tpu as pltpu
from jax.experimental.pallas import tpu_sc as plsc

N = 10000
E = 160000
D = 256
H = 128
MB = 400
NMB = N // MB
NS = 16
EPT = E // NS
CH = 128
NCHUNK = 80
EPTP = NCHUNK * CH
GSZ = 8
NGRP = NCHUNK // GSZ
NPAD = N + 8
CP_TILES = 10
CP_ROWS = N // CP_TILES


def _mesh():
    return plsc.VectorSubcoreMesh(
        core_axis_name="c", subcore_axis_name="s",
        num_cores=2, num_subcores=NS)


@functools.lru_cache(maxsize=None)
def _make_sc_segsum():
    return pl.kernel(
        _sc_segsum_body,
        out_type=(jax.ShapeDtypeStruct((2 * N, H), jnp.float32),
                  jax.ShapeDtypeStruct((2 * N, H), jnp.float32)),
        mesh=_mesh(),
        scratch_types=[
            pltpu.VMEM_SHARED((NPAD, H), jnp.float32),
            pltpu.VMEM((GSZ, CH), jnp.int32),
            [pltpu.VMEM((CH,), jnp.int32) for _ in range(GSZ)],
            pltpu.VMEM((CH, H), jnp.float32),
            pltpu.VMEM((CH, H), jnp.float32),
            pltpu.SemaphoreType.DMA,
            pltpu.SemaphoreType.DMA,
            pltpu.SemaphoreType.DMA,
        ],
    )


def _sc_segsum(tp, ta, row_g, col_g, row_s, col_s, zeros):
    return _make_sc_segsum()(tp, ta, row_g, col_g, row_s, col_s, zeros)


def _one_direction(c, s, w, table_ref, sidx_ref, didx_ref, zeros_ref,
                   out_ref, acc, idx2b, drefs, buf0, buf1, gsem, ssem, isem):
    @pl.when(s < CP_TILES)
    def _():
        pltpu.sync_copy(zeros_ref, acc.at[pl.ds(s * CP_ROWS, CP_ROWS)])

    plsc.subcore_barrier()
    bufs = (buf0, buf1)

    def drain(sem, buf):
        pltpu.make_async_copy(table_ref.at[pl.ds(0, CH)], buf, sem).wait()

    def group(g, carry):
        @pl.when(g >= 1)
        def _():
            drain(ssem, buf1)

        pltpu.sync_copy(sidx_ref.at[w, pl.ds(g * GSZ, GSZ)], idx2b)
        for jj in range(GSZ):
            pltpu.async_copy(didx_ref.at[s, g * GSZ + jj], drefs[jj], isem)
        pltpu.async_copy(table_ref.at[idx2b.at[0]], buf0, gsem)
        for jj in range(GSZ):
            pltpu.make_async_copy(didx_ref.at[s, 0], drefs[jj], isem).wait()
        for jj in range(GSZ):
            buf, nbuf = bufs[jj % 2], bufs[(jj + 1) % 2]
            if jj >= 1:
                drain(ssem, nbuf)
            if jj < GSZ - 1:
                pltpu.async_copy(table_ref.at[idx2b.at[jj + 1]], nbuf, gsem)
            drain(gsem, buf)
            pltpu.async_copy(buf, acc.at[drefs[jj]], ssem, add=True)
        return carry

    lax.fori_loop(0, NGRP, group, 0)
    drain(ssem, buf1)
    plsc.subcore_barrier()

    @pl.when(s < CP_TILES)
    def _():
        row0 = c * N + s * CP_ROWS
        pltpu.sync_copy(acc.at[pl.ds(s * CP_ROWS, CP_ROWS)],
                        out_ref.at[pl.ds(row0, CP_ROWS)])


def _sc_segsum_body(tp_ref, ta_ref, row_g_ref, col_g_ref, row_s_ref,
                    col_s_ref, zeros_ref, outp_ref, outa_ref,
                    acc, idx2b, drefs, buf0, buf1, gsem, ssem, isem):
    c = lax.axis_index("c")
    s = lax.axis_index("s")
    w = c * NS + s
    _one_direction(c, s, w, tp_ref, row_g_ref, col_s_ref, zeros_ref,
                   outp_ref, acc, idx2b, drefs, buf0, buf1, gsem, ssem, isem)
    plsc.subcore_barrier()
    _one_direction(c, s, w, ta_ref, col_g_ref, row_s_ref, zeros_ref,
                   outa_ref, acc, idx2b, drefs, buf0, buf1, gsem, ssem, isem)


@functools.lru_cache(maxsize=None)
def _make_sc_counts():
    return pl.kernel(
        _sc_counts_body,
        out_type=jax.ShapeDtypeStruct((2 * N, H), jnp.float32),
        mesh=_mesh(),
        scratch_types=[
            pltpu.VMEM_SHARED((NPAD, H), jnp.float32),
            pltpu.VMEM((CH,), jnp.int32),
            pltpu.VMEM((CH, H), jnp.float32),
        ],
    )


def _sc_counts(didx2, zeros, ones):
    return _make_sc_counts()(didx2, zeros, ones)


def _sc_counts_body(didx_ref, zeros_ref, ones_ref, out_ref, cacc, didx_vm,
                    ones_v):
    c = lax.axis_index("c")
    s = lax.axis_index("s")

    @pl.when(s < CP_TILES)
    def _():
        pltpu.sync_copy(zeros_ref, cacc.at[pl.ds(s * CP_ROWS, CP_ROWS)])

    pltpu.sync_copy(ones_ref, ones_v)
    plsc.subcore_barrier()

    def body(i, carry):
        pltpu.sync_copy(didx_ref.at[c * NS + s, i], didx_vm)
        pltpu.sync_copy(ones_v, cacc.at[didx_vm], add=True)
        return carry

    lax.fori_loop(0, NCHUNK, body, 0)
    plsc.subcore_barrier()

    @pl.when(s < CP_TILES)
    def _():
        row0 = c * N + s * CP_ROWS
        pltpu.sync_copy(cacc.at[pl.ds(s * CP_ROWS, CP_ROWS)],
                        out_ref.at[pl.ds(row0, CP_ROWS)])


def _proj_body(x_ref, w_ref, b_ref, o_ref):
    h = jnp.dot(x_ref[...], w_ref[...], preferred_element_type=jnp.float32)
    o_ref[...] = jnp.maximum(h + b_ref[...], 0.0)


def _tc_proj(x, w, b):
    return pl.pallas_call(
        _proj_body,
        grid=(NMB,),
        in_specs=[
            pl.BlockSpec((MB, D), lambda m: (m, 0)),
            pl.BlockSpec((D, D), lambda m: (0, 0)),
            pl.BlockSpec((1, D), lambda m: (0, 0)),
        ],
        out_specs=pl.BlockSpec((MB, D), lambda m: (m, 0)),
        out_shape=jax.ShapeDtypeStruct((N, D), jnp.float32),
    )(x, w, b)


def _post_body(norm_relu, acc_ref, cnt_ref, xd_ref, wl_ref, bl_ref, wr_ref,
               o_ref):
    denom = jnp.maximum(cnt_ref[...], 1.0)
    a0 = acc_ref[0] / denom
    a1 = acc_ref[1] / denom
    out = (jnp.dot(a0, wl_ref[0], preferred_element_type=jnp.float32)
           + jnp.dot(a1, wl_ref[1], preferred_element_type=jnp.float32)
           + jnp.dot(xd_ref[...], wr_ref[...],
                     preferred_element_type=jnp.float32)
           + bl_ref[...])
    if norm_relu:
        n = jnp.sqrt(jnp.sum(out * out, axis=1, keepdims=True))
        out = out / jnp.maximum(n, 1e-12)
        out = jnp.maximum(out, 0.0)
    o_ref[...] = out


def _tc_post(acc, cnt, xd, wl, bl, wr, norm_relu):
    return pl.pallas_call(
        functools.partial(_post_body, norm_relu),
        grid=(NMB,),
        in_specs=[
            pl.BlockSpec((2, MB, H), lambda m: (0, m, 0)),
            pl.BlockSpec((MB, 1), lambda m: (m, 0)),
            pl.BlockSpec((MB, D), lambda m: (m, 0)),
            pl.BlockSpec((2, H, D), lambda m: (0, 0, 0)),
            pl.BlockSpec((1, D), lambda m: (0, 0)),
            pl.BlockSpec((D, D), lambda m: (0, 0)),
        ],
        out_specs=pl.BlockSpec((MB, D), lambda m: (m, 0)),
        out_shape=jax.ShapeDtypeStruct((N, D), jnp.float32),
    )(acc, cnt, xd, wl, bl, wr)


def kernel(x_author, x_paper, edge_index, W_proj, b_proj, W_l, b_l, W_r):
    row2 = edge_index[0].reshape(NS, EPT)
    col2 = edge_index[1].reshape(NS, EPT)
    pad = EPTP - EPT
    row_gp = jnp.pad(row2, ((0, 0), (0, pad)))
    col_gp = jnp.pad(col2, ((0, 0), (0, pad)))
    row_g = jnp.stack([row_gp * 2, row_gp * 2 + 1]).reshape(
        2 * NS, NCHUNK, CH)
    col_g = jnp.stack([col_gp * 2, col_gp * 2 + 1]).reshape(
        2 * NS, NCHUNK, CH)
    row_s = jnp.pad(row2, ((0, 0), (0, pad)),
                    constant_values=N).reshape(NS, NCHUNK, CH)
    col_s = jnp.pad(col2, ((0, 0), (0, pad)),
                    constant_values=N).reshape(NS, NCHUNK, CH)
    zeros_h = jnp.zeros((CP_ROWS, H), jnp.float32)
    ones_h = jnp.ones((CH, H), jnp.float32)

    didx2 = jnp.concatenate([col_s, row_s], axis=0)
    counts = _sc_counts(didx2, zeros_h, ones_h)
    cnt_p = counts[0:N, 0:1]
    cnt_a = counts[N:2 * N, 0:1]

    def seg_both(ta_for_p, tp_for_a):
        op, oa = _sc_segsum(ta_for_p.reshape(2 * N, H),
                            tp_for_a.reshape(2 * N, H),
                            row_g, col_g, row_s, col_s, zeros_h)
        return op.reshape(2, N, H), oa.reshape(2, N, H)

    xa, xp = x_author, x_paper
    for i in range(4):
        ha = _tc_proj(xa, W_proj[i, 0], b_proj[i, 0].reshape(1, D))
        hp = _tc_proj(xp, W_proj[i, 1], b_proj[i, 1].reshape(1, D))
        sp, sa = seg_both(ha, hp)
        xp_new = _tc_post(sp, cnt_p, xp, W_l[i, 0].reshape(2, H, D),
                          b_l[i, 0].reshape(1, D), W_r[i, 0], True)
        xa_new = _tc_post(sa, cnt_a, xa, W_l[i, 1].reshape(2, H, D),
                          b_l[i, 1].reshape(1, D), W_r[i, 1], True)
        xp, xa = xp_new, xa_new

    sp, sa = seg_both(xa, xp)
    out_p = _tc_post(sp, cnt_p, xp, W_l[4, 0].reshape(2, H, D),
                     b_l[4, 0].reshape(1, D), W_r[4, 0], False)
    out_a = _tc_post(sa, cnt_a, xa, W_l[4, 1].reshape(2, H, D),
                     b_l[4, 1].reshape(1, D), W_r[4, 1], False)
    return (out_a, out_p)


# speedup vs baseline: 2.1431x; 1.5509x over previous
import functools

import jax
import jax.numpy as jnp
from jax import lax
from jax.experimental import pallas as pl
from jax.experimental.pallas import tpu as pltpu
from jax.experimental.pallas import tpu_sc as plsc

N = 10000
E = 160000
D = 256
H = 128
MB = 400
NMB = N // MB
NS = 16
EPT = E // NS
CH = 112
NCHUNK = 90
EPTP = NCHUNK * CH
GSZ = 6
NGRP = NCHUNK // GSZ
NPAD = N + 8
CP_TILES = 10
CP_ROWS = N // CP_TILES


def _mesh():
    return plsc.VectorSubcoreMesh(
        core_axis_name="c", subcore_axis_name="s",
        num_cores=2, num_subcores=NS)


@functools.lru_cache(maxsize=None)
def _make_sc_segsum():
    return pl.kernel(
        _sc_segsum_body,
        out_type=(jax.ShapeDtypeStruct((2 * N, H), jnp.float32),
                  jax.ShapeDtypeStruct((2 * N, H), jnp.float32)),
        mesh=_mesh(),
        scratch_types=[
            pltpu.VMEM_SHARED((NPAD, H), jnp.float32),
            pltpu.VMEM((2, GSZ, CH), jnp.int32),
            [pltpu.VMEM((CH,), jnp.int32) for _ in range(GSZ)],
            [pltpu.VMEM((CH, H), jnp.float32) for _ in range(3)],
            pltpu.SemaphoreType.DMA,
            pltpu.SemaphoreType.DMA,
            pltpu.SemaphoreType.DMA,
            pltpu.SemaphoreType.DMA,
        ],
    )


def _sc_segsum(tp, ta, row_g, col_g, row_s, col_s, zeros):
    return _make_sc_segsum()(tp, ta, row_g, col_g, row_s, col_s, zeros)


def _one_direction(c, s, w, table_ref, sidx_ref, didx_ref, zeros_ref,
                   out_ref, acc, idx2b, drefs, bufs, gsem, ssem, isem, i2sem):
    @pl.when(s < CP_TILES)
    def _():
        pltpu.sync_copy(zeros_ref, acc.at[pl.ds(s * CP_ROWS, CP_ROWS)])

    plsc.subcore_barrier()

    def drain(sem, buf):
        pltpu.make_async_copy(table_ref.at[pl.ds(0, CH)], buf, sem).wait()

    def drain_idx(sem, ref):
        pltpu.make_async_copy(didx_ref.at[s, 0], ref, sem).wait()

    pltpu.sync_copy(sidx_ref.at[w, 0], idx2b.at[0])
    pltpu.async_copy(table_ref.at[idx2b.at[0, 0]], bufs[0], gsem)
    pltpu.async_copy(table_ref.at[idx2b.at[0, 1]], bufs[1], gsem)

    def group(g, carry):
        gp = g % 2

        @pl.when(g >= 1)
        def _():
            drain(ssem, bufs[2])

        @pl.when(g + 1 < NGRP)
        def _():
            pltpu.async_copy(sidx_ref.at[w, g + 1],
                             idx2b.at[(g + 1) % 2], i2sem)

        for jj in range(GSZ):
            pltpu.async_copy(didx_ref.at[s, g * GSZ + jj], drefs[jj], isem)
        for jj in range(GSZ):
            drain_idx(isem, drefs[jj])

        for jj in range(GSZ):
            j = g * GSZ + jj
            buf = bufs[jj % 3]
            fbuf = bufs[(jj + 2) % 3]

            if jj >= 1:
                drain(ssem, fbuf)

            if jj < GSZ - 2:
                @pl.when(j + 2 < NCHUNK)
                def _():
                    pltpu.async_copy(table_ref.at[idx2b.at[gp, jj + 2]],
                                     fbuf, gsem)
            else:
                @pl.when(j + 2 < NCHUNK)
                def _():
                    if jj == GSZ - 2:
                        pltpu.make_async_copy(sidx_ref.at[w, 0],
                                              idx2b.at[0], i2sem).wait()
                    pltpu.async_copy(
                        table_ref.at[idx2b.at[1 - gp, jj + 2 - GSZ]],
                        fbuf, gsem)

            drain(gsem, buf)
            pltpu.async_copy(buf, acc.at[drefs[jj]], ssem, add=True)
        return carry

    lax.fori_loop(0, NGRP, group, 0)
    drain(ssem, bufs[2])
    plsc.subcore_barrier()

    @pl.when(s < CP_TILES)
    def _():
        row0 = c * N + s * CP_ROWS
        pltpu.sync_copy(acc.at[pl.ds(s * CP_ROWS, CP_ROWS)],
                        out_ref.at[pl.ds(row0, CP_ROWS)])


def _sc_segsum_body(tp_ref, ta_ref, row_g_ref, col_g_ref, row_s_ref,
                    col_s_ref, zeros_ref, outp_ref, outa_ref,
                    acc, idx2b, drefs, bufs, gsem, ssem, isem, i2sem):
    c = lax.axis_index("c")
    s = lax.axis_index("s")
    w = c * NS + s
    _one_direction(c, s, w, tp_ref, row_g_ref, col_s_ref, zeros_ref,
                   outp_ref, acc, idx2b, drefs, bufs, gsem, ssem, isem, i2sem)
    plsc.subcore_barrier()
    _one_direction(c, s, w, ta_ref, col_g_ref, row_s_ref, zeros_ref,
                   outa_ref, acc, idx2b, drefs, bufs, gsem, ssem, isem, i2sem)


@functools.lru_cache(maxsize=None)
def _make_sc_counts():
    return pl.kernel(
        _sc_counts_body,
        out_type=jax.ShapeDtypeStruct((2 * N, H), jnp.float32),
        mesh=_mesh(),
        scratch_types=[
            pltpu.VMEM_SHARED((NPAD, H), jnp.float32),
            pltpu.VMEM((CH,), jnp.int32),
            pltpu.VMEM((CH, H), jnp.float32),
        ],
    )


def _sc_counts(didx2, zeros, ones):
    return _make_sc_counts()(didx2, zeros, ones)


def _sc_counts_body(didx_ref, zeros_ref, ones_ref, out_ref, cacc, didx_vm,
                    ones_v):
    c = lax.axis_index("c")
    s = lax.axis_index("s")

    @pl.when(s < CP_TILES)
    def _():
        pltpu.sync_copy(zeros_ref, cacc.at[pl.ds(s * CP_ROWS, CP_ROWS)])

    pltpu.sync_copy(ones_ref, ones_v)
    plsc.subcore_barrier()

    def body(i, carry):
        pltpu.sync_copy(didx_ref.at[c * NS + s, i], didx_vm)
        pltpu.sync_copy(ones_v, cacc.at[didx_vm], add=True)
        return carry

    lax.fori_loop(0, NCHUNK, body, 0)
    plsc.subcore_barrier()

    @pl.when(s < CP_TILES)
    def _():
        row0 = c * N + s * CP_ROWS
        pltpu.sync_copy(cacc.at[pl.ds(s * CP_ROWS, CP_ROWS)],
                        out_ref.at[pl.ds(row0, CP_ROWS)])


def _proj_body(x_ref, w_ref, b_ref, o_ref):
    h = jnp.dot(x_ref[...], w_ref[...], preferred_element_type=jnp.float32)
    o_ref[...] = jnp.maximum(h + b_ref[...], 0.0)


def _tc_proj(x, w, b):
    return pl.pallas_call(
        _proj_body,
        grid=(NMB,),
        in_specs=[
            pl.BlockSpec((MB, D), lambda m: (m, 0)),
            pl.BlockSpec((D, D), lambda m: (0, 0)),
            pl.BlockSpec((1, D), lambda m: (0, 0)),
        ],
        out_specs=pl.BlockSpec((MB, D), lambda m: (m, 0)),
        out_shape=jax.ShapeDtypeStruct((N, D), jnp.float32),
    )(x, w, b)


def _post_body(norm_relu, acc_ref, cnt_ref, xd_ref, wl_ref, bl_ref, wr_ref,
               o_ref):
    denom = jnp.maximum(cnt_ref[...], 1.0)
    a0 = acc_ref[0] / denom
    a1 = acc_ref[1] / denom
    out = (jnp.dot(a0, wl_ref[0], preferred_element_type=jnp.float32)
           + jnp.dot(a1, wl_ref[1], preferred_element_type=jnp.float32)
           + jnp.dot(xd_ref[...], wr_ref[...],
                     preferred_element_type=jnp.float32)
           + bl_ref[...])
    if norm_relu:
        n = jnp.sqrt(jnp.sum(out * out, axis=1, keepdims=True))
        out = out / jnp.maximum(n, 1e-12)
        out = jnp.maximum(out, 0.0)
    o_ref[...] = out


def _tc_post(acc, cnt, xd, wl, bl, wr, norm_relu):
    return pl.pallas_call(
        functools.partial(_post_body, norm_relu),
        grid=(NMB,),
        in_specs=[
            pl.BlockSpec((2, MB, H), lambda m: (0, m, 0)),
            pl.BlockSpec((MB, 1), lambda m: (m, 0)),
            pl.BlockSpec((MB, D), lambda m: (m, 0)),
            pl.BlockSpec((2, H, D), lambda m: (0, 0, 0)),
            pl.BlockSpec((1, D), lambda m: (0, 0)),
            pl.BlockSpec((D, D), lambda m: (0, 0)),
        ],
        out_specs=pl.BlockSpec((MB, D), lambda m: (m, 0)),
        out_shape=jax.ShapeDtypeStruct((N, D), jnp.float32),
    )(acc, cnt, xd, wl, bl, wr)


def kernel(x_author, x_paper, edge_index, W_proj, b_proj, W_l, b_l, W_r):
    row2 = edge_index[0].reshape(NS, EPT)
    col2 = edge_index[1].reshape(NS, EPT)
    pad = EPTP - EPT
    row_gp = jnp.pad(row2, ((0, 0), (0, pad)))
    col_gp = jnp.pad(col2, ((0, 0), (0, pad)))
    row_g = jnp.stack([row_gp * 2, row_gp * 2 + 1]).reshape(
        2 * NS, NGRP, GSZ, CH)
    col_g = jnp.stack([col_gp * 2, col_gp * 2 + 1]).reshape(
        2 * NS, NGRP, GSZ, CH)
    row_s = jnp.pad(row2, ((0, 0), (0, pad)),
                    constant_values=N).reshape(NS, NCHUNK, CH)
    col_s = jnp.pad(col2, ((0, 0), (0, pad)),
                    constant_values=N).reshape(NS, NCHUNK, CH)
    zeros_h = jnp.zeros((CP_ROWS, H), jnp.float32)
    ones_h = jnp.ones((CH, H), jnp.float32)

    didx2 = jnp.concatenate([col_s, row_s], axis=0)
    counts = _sc_counts(didx2, zeros_h, ones_h)
    cnt_p = counts[0:N, 0:1]
    cnt_a = counts[N:2 * N, 0:1]

    def seg_both(ta_for_p, tp_for_a):
        op, oa = _sc_segsum(ta_for_p.reshape(2 * N, H),
                            tp_for_a.reshape(2 * N, H),
                            row_g, col_g, row_s, col_s, zeros_h)
        return op.reshape(2, N, H), oa.reshape(2, N, H)

    xa, xp = x_author, x_paper
    for i in range(4):
        ha = _tc_proj(xa, W_proj[i, 0], b_proj[i, 0].reshape(1, D))
        hp = _tc_proj(xp, W_proj[i, 1], b_proj[i, 1].reshape(1, D))
        sp, sa = seg_both(ha, hp)
        xp_new = _tc_post(sp, cnt_p, xp, W_l[i, 0].reshape(2, H, D),
                          b_l[i, 0].reshape(1, D), W_r[i, 0], True)
        xa_new = _tc_post(sa, cnt_a, xa, W_l[i, 1].reshape(2, H, D),
                          b_l[i, 1].reshape(1, D), W_r[i, 1], True)
        xp, xa = xp_new, xa_new

    sp, sa = seg_both(xa, xp)
    out_p = _tc_post(sp, cnt_p, xp, W_l[4, 0].reshape(2, H, D),
                     b_l[4, 0].reshape(1, D), W_r[4, 0], False)
    out_a = _tc_post(sa, cnt_a, xa, W_l[4, 1].reshape(2, H, D),
                     b_l[4, 1].reshape(1, D), W_r[4, 1], False)
    return (out_a, out_p)


# speedup vs baseline: 2.1843x; 1.0192x over previous
import functools

import jax
import jax.numpy as jnp
from jax import lax
from jax.experimental import pallas as pl
from jax.experimental.pallas import tpu as pltpu
from jax.experimental.pallas import tpu_sc as plsc

N = 10000
E = 160000
D = 256
H = 128
MB = 400
NMB = N // MB
NS = 16
EPT = E // NS
CH = 112
NCHUNK = 90
EPTP = NCHUNK * CH
GSZ = 6
NGRP = NCHUNK // GSZ
NPAD = N + 8
CP_TILES = 10
CP_ROWS = N // CP_TILES


def _mesh():
    return plsc.VectorSubcoreMesh(
        core_axis_name="c", subcore_axis_name="s",
        num_cores=2, num_subcores=NS)


@functools.lru_cache(maxsize=None)
def _make_sc_segsum():
    return pl.kernel(
        _sc_segsum_body,
        out_type=(jax.ShapeDtypeStruct((2 * N, H), jnp.float32),
                  jax.ShapeDtypeStruct((2 * N, H), jnp.float32)),
        mesh=_mesh(),
        scratch_types=[
            pltpu.VMEM_SHARED((NPAD, H), jnp.float32),
            pltpu.VMEM((2, GSZ, CH), jnp.int32),
            [pltpu.VMEM((CH,), jnp.int32) for _ in range(GSZ)],
            [pltpu.VMEM((CH, H), jnp.float32) for _ in range(3)],
            pltpu.SemaphoreType.DMA,
            pltpu.SemaphoreType.DMA,
            pltpu.SemaphoreType.DMA,
            pltpu.SemaphoreType.DMA,
        ],
    )


def _sc_segsum(tp, ta, row_g, col_g, row_s, col_s, zeros):
    return _make_sc_segsum()(tp, ta, row_g, col_g, row_s, col_s, zeros)


def _one_direction(c, s, w, table_ref, sidx_ref, didx_ref, zeros_ref,
                   out_ref, acc, idx2b, drefs, bufs, gsem, ssem, isem, i2sem):
    @pl.when(s < CP_TILES)
    def _():
        pltpu.sync_copy(zeros_ref, acc.at[pl.ds(s * CP_ROWS, CP_ROWS)])

    plsc.subcore_barrier()

    def drain(sem, buf):
        pltpu.make_async_copy(table_ref.at[pl.ds(0, CH)], buf, sem).wait()

    def drain_idx(sem, ref):
        pltpu.make_async_copy(didx_ref.at[s, 0], ref, sem).wait()

    pltpu.sync_copy(sidx_ref.at[w, 0], idx2b.at[0])
    pltpu.async_copy(table_ref.at[idx2b.at[0, 0]], bufs[0], gsem)
    pltpu.async_copy(table_ref.at[idx2b.at[0, 1]], bufs[1], gsem)

    def group(g, carry):
        gp = g % 2

        @pl.when(g >= 1)
        def _():
            drain(ssem, bufs[2])

        @pl.when(g + 1 < NGRP)
        def _():
            pltpu.async_copy(sidx_ref.at[w, g + 1],
                             idx2b.at[(g + 1) % 2], i2sem)

        for jj in range(GSZ):
            pltpu.async_copy(didx_ref.at[s, g * GSZ + jj], drefs[jj], isem)
        for jj in range(GSZ):
            drain_idx(isem, drefs[jj])

        for jj in range(GSZ):
            j = g * GSZ + jj
            buf = bufs[jj % 3]
            fbuf = bufs[(jj + 2) % 3]

            if jj >= 1:
                drain(ssem, fbuf)

            if jj < GSZ - 2:
                @pl.when(j + 2 < NCHUNK)
                def _():
                    pltpu.async_copy(table_ref.at[idx2b.at[gp, jj + 2]],
                                     fbuf, gsem)
            else:
                @pl.when(j + 2 < NCHUNK)
                def _():
                    if jj == GSZ - 2:
                        pltpu.make_async_copy(sidx_ref.at[w, 0],
                                              idx2b.at[0], i2sem).wait()
                    pltpu.async_copy(
                        table_ref.at[idx2b.at[1 - gp, jj + 2 - GSZ]],
                        fbuf, gsem)

            drain(gsem, buf)
            pltpu.async_copy(buf, acc.at[drefs[jj]], ssem, add=True)
        return carry

    lax.fori_loop(0, NGRP, group, 0)
    drain(ssem, bufs[2])
    plsc.subcore_barrier()

    @pl.when(s < CP_TILES)
    def _():
        row0 = c * N + s * CP_ROWS
        pltpu.sync_copy(acc.at[pl.ds(s * CP_ROWS, CP_ROWS)],
                        out_ref.at[pl.ds(row0, CP_ROWS)])


def _sc_segsum_body(tp_ref, ta_ref, row_g_ref, col_g_ref, row_s_ref,
                    col_s_ref, zeros_ref, outp_ref, outa_ref,
                    acc, idx2b, drefs, bufs, gsem, ssem, isem, i2sem):
    c = lax.axis_index("c")
    s = lax.axis_index("s")
    w = c * NS + s
    _one_direction(c, s, w, tp_ref, row_g_ref, col_s_ref, zeros_ref,
                   outp_ref, acc, idx2b, drefs, bufs, gsem, ssem, isem, i2sem)
    plsc.subcore_barrier()
    _one_direction(c, s, w, ta_ref, col_g_ref, row_s_ref, zeros_ref,
                   outa_ref, acc, idx2b, drefs, bufs, gsem, ssem, isem, i2sem)


@functools.lru_cache(maxsize=None)
def _make_sc_counts():
    return pl.kernel(
        _sc_counts_body,
        out_type=jax.ShapeDtypeStruct((2 * N, H), jnp.float32),
        mesh=_mesh(),
        scratch_types=[
            pltpu.VMEM_SHARED((NPAD, H), jnp.float32),
            pltpu.VMEM((CH,), jnp.int32),
            pltpu.VMEM((CH,), jnp.int32),
            pltpu.VMEM((CH, H), jnp.float32),
            pltpu.SemaphoreType.DMA,
            pltpu.SemaphoreType.DMA,
        ],
    )


def _sc_counts(didx2, zeros, ones):
    return _make_sc_counts()(didx2, zeros, ones)


def _sc_counts_body(didx_ref, zeros_ref, ones_ref, out_ref, cacc, didx0,
                    didx1, ones_v, isem, ssem):
    c = lax.axis_index("c")
    s = lax.axis_index("s")
    ws = c * NS + s

    @pl.when(s < CP_TILES)
    def _():
        pltpu.sync_copy(zeros_ref, cacc.at[pl.ds(s * CP_ROWS, CP_ROWS)])

    pltpu.sync_copy(ones_ref, ones_v)
    plsc.subcore_barrier()
    refs = (didx0, didx1)
    pltpu.async_copy(didx_ref.at[ws, 0], didx0, isem)

    def drain_idx(ref):
        pltpu.make_async_copy(didx_ref.at[ws, 0], ref, isem).wait()

    def drain_sc(ref):
        pltpu.make_async_copy(zeros_ref.at[pl.ds(0, CH)], ones_v, ssem).wait()

    def body(g, carry):
        for par in range(2):
            i = g * 2 + par

            @pl.when(i >= 1)
            def _():
                drain_sc(None)

            @pl.when(i + 1 < NCHUNK)
            def _():
                pltpu.async_copy(didx_ref.at[ws, i + 1], refs[1 - par], isem)

            drain_idx(refs[par])
            pltpu.async_copy(ones_v, cacc.at[refs[par]], ssem, add=True)
        return carry

    lax.fori_loop(0, NCHUNK // 2, body, 0)
    drain_sc(None)
    plsc.subcore_barrier()

    @pl.when(s < CP_TILES)
    def _():
        row0 = c * N + s * CP_ROWS
        pltpu.sync_copy(cacc.at[pl.ds(s * CP_ROWS, CP_ROWS)],
                        out_ref.at[pl.ds(row0, CP_ROWS)])


def _proj_body(x_ref, w_ref, b_ref, o_ref):
    h = jnp.dot(x_ref[...], w_ref[...], preferred_element_type=jnp.float32)
    o_ref[...] = jnp.maximum(h + b_ref[...], 0.0)


def _tc_proj(x, w, b):
    return pl.pallas_call(
        _proj_body,
        grid=(NMB,),
        in_specs=[
            pl.BlockSpec((MB, D), lambda m: (m, 0)),
            pl.BlockSpec((D, D), lambda m: (0, 0)),
            pl.BlockSpec((1, D), lambda m: (0, 0)),
        ],
        out_specs=pl.BlockSpec((MB, D), lambda m: (m, 0)),
        out_shape=jax.ShapeDtypeStruct((N, D), jnp.float32),
    )(x, w, b)


def _post_body(norm_relu, acc_ref, cnt_ref, xd_ref, wl_ref, bl_ref, wr_ref,
               o_ref):
    denom = jnp.maximum(cnt_ref[...], 1.0)
    a0 = acc_ref[0] / denom
    a1 = acc_ref[1] / denom
    out = (jnp.dot(a0, wl_ref[0], preferred_element_type=jnp.float32)
           + jnp.dot(a1, wl_ref[1], preferred_element_type=jnp.float32)
           + jnp.dot(xd_ref[...], wr_ref[...],
                     preferred_element_type=jnp.float32)
           + bl_ref[...])
    if norm_relu:
        n = jnp.sqrt(jnp.sum(out * out, axis=1, keepdims=True))
        out = out / jnp.maximum(n, 1e-12)
        out = jnp.maximum(out, 0.0)
    o_ref[...] = out


def _tc_post(acc, cnt, xd, wl, bl, wr, norm_relu):
    return pl.pallas_call(
        functools.partial(_post_body, norm_relu),
        grid=(NMB,),
        in_specs=[
            pl.BlockSpec((2, MB, H), lambda m: (0, m, 0)),
            pl.BlockSpec((MB, 1), lambda m: (m, 0)),
            pl.BlockSpec((MB, D), lambda m: (m, 0)),
            pl.BlockSpec((2, H, D), lambda m: (0, 0, 0)),
            pl.BlockSpec((1, D), lambda m: (0, 0)),
            pl.BlockSpec((D, D), lambda m: (0, 0)),
        ],
        out_specs=pl.BlockSpec((MB, D), lambda m: (m, 0)),
        out_shape=jax.ShapeDtypeStruct((N, D), jnp.float32),
    )(acc, cnt, xd, wl, bl, wr)


def kernel(x_author, x_paper, edge_index, W_proj, b_proj, W_l, b_l, W_r):
    row2 = edge_index[0].reshape(NS, EPT)
    col2 = edge_index[1].reshape(NS, EPT)
    pad = EPTP - EPT
    row_gp = jnp.pad(row2, ((0, 0), (0, pad)))
    col_gp = jnp.pad(col2, ((0, 0), (0, pad)))
    row_g = jnp.stack([row_gp * 2, row_gp * 2 + 1]).reshape(
        2 * NS, NGRP, GSZ, CH)
    col_g = jnp.stack([col_gp * 2, col_gp * 2 + 1]).reshape(
        2 * NS, NGRP, GSZ, CH)
    row_s = jnp.pad(row2, ((0, 0), (0, pad)),
                    constant_values=N).reshape(NS, NCHUNK, CH)
    col_s = jnp.pad(col2, ((0, 0), (0, pad)),
                    constant_values=N).reshape(NS, NCHUNK, CH)
    zeros_h = jnp.zeros((CP_ROWS, H), jnp.float32)
    ones_h = jnp.ones((CH, H), jnp.float32)

    didx2 = jnp.concatenate([col_s, row_s], axis=0)
    counts = _sc_counts(didx2, zeros_h, ones_h)
    cnt_p = counts[0:N, 0:1]
    cnt_a = counts[N:2 * N, 0:1]

    def seg_both(ta_for_p, tp_for_a):
        op, oa = _sc_segsum(ta_for_p.reshape(2 * N, H),
                            tp_for_a.reshape(2 * N, H),
                            row_g, col_g, row_s, col_s, zeros_h)
        return op.reshape(2, N, H), oa.reshape(2, N, H)

    xa, xp = x_author, x_paper
    for i in range(4):
        ha = _tc_proj(xa, W_proj[i, 0], b_proj[i, 0].reshape(1, D))
        hp = _tc_proj(xp, W_proj[i, 1], b_proj[i, 1].reshape(1, D))
        sp, sa = seg_both(ha, hp)
        xp_new = _tc_post(sp, cnt_p, xp, W_l[i, 0].reshape(2, H, D),
                          b_l[i, 0].reshape(1, D), W_r[i, 0], True)
        xa_new = _tc_post(sa, cnt_a, xa, W_l[i, 1].reshape(2, H, D),
                          b_l[i, 1].reshape(1, D), W_r[i, 1], True)
        xp, xa = xp_new, xa_new

    sp, sa = seg_both(xa, xp)
    out_p = _tc_post(sp, cnt_p, xp, W_l[4, 0].reshape(2, H, D),
                     b_l[4, 0].reshape(1, D), W_r[4, 0], False)
    out_a = _tc_post(sa, cnt_a, xa, W_l[4, 1].reshape(2, H, D),
                     b_l[4, 1].reshape(1, D), W_r[4, 1], False)
    return (out_a, out_p)


# speedup vs baseline: 2.3418x; 1.0721x over previous
import functools

import jax
import jax.numpy as jnp
from jax import lax
from jax.experimental import pallas as pl
from jax.experimental.pallas import tpu as pltpu
from jax.experimental.pallas import tpu_sc as plsc

N = 10000
E = 160000
D = 256
H = 128
MB = 1000
NMB = N // MB
NS = 16
EPT = E // NS
CH = 112
NCHUNK = 90
EPTP = NCHUNK * CH
GSZ = 6
NGRP = NCHUNK // GSZ
NPAD = N + 8
CP_TILES = 10
CP_ROWS = N // CP_TILES


def _mesh():
    return plsc.VectorSubcoreMesh(
        core_axis_name="c", subcore_axis_name="s",
        num_cores=2, num_subcores=NS)


@functools.lru_cache(maxsize=None)
def _make_sc_segsum():
    return pl.kernel(
        _sc_segsum_body,
        out_type=(jax.ShapeDtypeStruct((2 * N, H), jnp.float32),
                  jax.ShapeDtypeStruct((2 * N, H), jnp.float32)),
        mesh=_mesh(),
        scratch_types=[
            pltpu.VMEM_SHARED((NPAD, H), jnp.float32),
            pltpu.VMEM((2, GSZ, CH), jnp.int32),
            [pltpu.VMEM((CH,), jnp.int32) for _ in range(GSZ)],
            [pltpu.VMEM((CH, H), jnp.float32) for _ in range(3)],
            pltpu.SemaphoreType.DMA,
            pltpu.SemaphoreType.DMA,
            pltpu.SemaphoreType.DMA,
            pltpu.SemaphoreType.DMA,
        ],
    )


def _sc_segsum(tp, ta, row_g, col_g, row_s, col_s, zeros):
    return _make_sc_segsum()(tp, ta, row_g, col_g, row_s, col_s, zeros)


def _one_direction(c, s, w, table_ref, sidx_ref, didx_ref, zeros_ref,
                   out_ref, acc, idx2b, drefs, bufs, gsem, ssem, isem, i2sem):
    @pl.when(s < CP_TILES)
    def _():
        pltpu.sync_copy(zeros_ref, acc.at[pl.ds(s * CP_ROWS, CP_ROWS)])

    plsc.subcore_barrier()

    def drain(sem, buf):
        pltpu.make_async_copy(table_ref.at[pl.ds(0, CH)], buf, sem).wait()

    def drain_idx(sem, ref):
        pltpu.make_async_copy(didx_ref.at[s, 0], ref, sem).wait()

    pltpu.sync_copy(sidx_ref.at[w, 0], idx2b.at[0])
    pltpu.async_copy(table_ref.at[idx2b.at[0, 0]], bufs[0], gsem)
    pltpu.async_copy(table_ref.at[idx2b.at[0, 1]], bufs[1], gsem)

    def group(g, carry):
        gp = g % 2

        @pl.when(g >= 1)
        def _():
            drain(ssem, bufs[2])

        @pl.when(g + 1 < NGRP)
        def _():
            pltpu.async_copy(sidx_ref.at[w, g + 1],
                             idx2b.at[(g + 1) % 2], i2sem)

        for jj in range(GSZ):
            pltpu.async_copy(didx_ref.at[s, g * GSZ + jj], drefs[jj], isem)
        for jj in range(GSZ):
            drain_idx(isem, drefs[jj])

        for jj in range(GSZ):
            j = g * GSZ + jj
            buf = bufs[jj % 3]
            fbuf = bufs[(jj + 2) % 3]

            if jj >= 1:
                drain(ssem, fbuf)

            if jj < GSZ - 2:
                @pl.when(j + 2 < NCHUNK)
                def _():
                    pltpu.async_copy(table_ref.at[idx2b.at[gp, jj + 2]],
                                     fbuf, gsem)
            else:
                @pl.when(j + 2 < NCHUNK)
                def _():
                    if jj == GSZ - 2:
                        pltpu.make_async_copy(sidx_ref.at[w, 0],
                                              idx2b.at[0], i2sem).wait()
                    pltpu.async_copy(
                        table_ref.at[idx2b.at[1 - gp, jj + 2 - GSZ]],
                        fbuf, gsem)

            drain(gsem, buf)
            pltpu.async_copy(buf, acc.at[drefs[jj]], ssem, add=True)
        return carry

    lax.fori_loop(0, NGRP, group, 0)
    drain(ssem, bufs[2])
    plsc.subcore_barrier()

    @pl.when(s < CP_TILES)
    def _():
        row0 = c * N + s * CP_ROWS
        pltpu.sync_copy(acc.at[pl.ds(s * CP_ROWS, CP_ROWS)],
                        out_ref.at[pl.ds(row0, CP_ROWS)])


def _sc_segsum_body(tp_ref, ta_ref, row_g_ref, col_g_ref, row_s_ref,
                    col_s_ref, zeros_ref, outp_ref, outa_ref,
                    acc, idx2b, drefs, bufs, gsem, ssem, isem, i2sem):
    c = lax.axis_index("c")
    s = lax.axis_index("s")
    w = c * NS + s
    _one_direction(c, s, w, tp_ref, row_g_ref, col_s_ref, zeros_ref,
                   outp_ref, acc, idx2b, drefs, bufs, gsem, ssem, isem, i2sem)
    plsc.subcore_barrier()
    _one_direction(c, s, w, ta_ref, col_g_ref, row_s_ref, zeros_ref,
                   outa_ref, acc, idx2b, drefs, bufs, gsem, ssem, isem, i2sem)


@functools.lru_cache(maxsize=None)
def _make_sc_counts():
    return pl.kernel(
        _sc_counts_body,
        out_type=jax.ShapeDtypeStruct((2 * N, H), jnp.float32),
        mesh=_mesh(),
        scratch_types=[
            pltpu.VMEM_SHARED((NPAD, H), jnp.float32),
            pltpu.VMEM((CH,), jnp.int32),
            pltpu.VMEM((CH,), jnp.int32),
            pltpu.VMEM((CH, H), jnp.float32),
            pltpu.SemaphoreType.DMA,
            pltpu.SemaphoreType.DMA,
        ],
    )


def _sc_counts(didx2, zeros, ones):
    return _make_sc_counts()(didx2, zeros, ones)


def _sc_counts_body(didx_ref, zeros_ref, ones_ref, out_ref, cacc, didx0,
                    didx1, ones_v, isem, ssem):
    c = lax.axis_index("c")
    s = lax.axis_index("s")
    ws = c * NS + s

    @pl.when(s < CP_TILES)
    def _():
        pltpu.sync_copy(zeros_ref, cacc.at[pl.ds(s * CP_ROWS, CP_ROWS)])

    pltpu.sync_copy(ones_ref, ones_v)
    plsc.subcore_barrier()
    refs = (didx0, didx1)
    pltpu.async_copy(didx_ref.at[ws, 0], didx0, isem)

    def drain_idx(ref):
        pltpu.make_async_copy(didx_ref.at[ws, 0], ref, isem).wait()

    def drain_sc(ref):
        pltpu.make_async_copy(zeros_ref.at[pl.ds(0, CH)], ones_v, ssem).wait()

    def body(g, carry):
        for par in range(2):
            i = g * 2 + par

            @pl.when(i >= 1)
            def _():
                drain_sc(None)

            @pl.when(i + 1 < NCHUNK)
            def _():
                pltpu.async_copy(didx_ref.at[ws, i + 1], refs[1 - par], isem)

            drain_idx(refs[par])
            pltpu.async_copy(ones_v, cacc.at[refs[par]], ssem, add=True)
        return carry

    lax.fori_loop(0, NCHUNK // 2, body, 0)
    drain_sc(None)
    plsc.subcore_barrier()

    @pl.when(s < CP_TILES)
    def _():
        row0 = c * N + s * CP_ROWS
        pltpu.sync_copy(cacc.at[pl.ds(s * CP_ROWS, CP_ROWS)],
                        out_ref.at[pl.ds(row0, CP_ROWS)])


def _proj_body(x_ref, w_ref, b_ref, o_ref):
    h = jnp.dot(x_ref[...], w_ref[...], preferred_element_type=jnp.float32)
    o_ref[...] = jnp.maximum(h + b_ref[...], 0.0)


def _tc_proj(x, w, b):
    return pl.pallas_call(
        _proj_body,
        grid=(NMB,),
        in_specs=[
            pl.BlockSpec((MB, D), lambda m: (m, 0)),
            pl.BlockSpec((D, D), lambda m: (0, 0)),
            pl.BlockSpec((1, D), lambda m: (0, 0)),
        ],
        out_specs=pl.BlockSpec((MB, D), lambda m: (m, 0)),
        out_shape=jax.ShapeDtypeStruct((N, D), jnp.float32),
    )(x, w, b)


def _post_body(norm_relu, acc_ref, cnt_ref, xd_ref, wl_ref, bl_ref, wr_ref,
               o_ref):
    denom = jnp.maximum(cnt_ref[...], 1.0)
    a0 = acc_ref[0] / denom
    a1 = acc_ref[1] / denom
    out = (jnp.dot(a0, wl_ref[0], preferred_element_type=jnp.float32)
           + jnp.dot(a1, wl_ref[1], preferred_element_type=jnp.float32)
           + jnp.dot(xd_ref[...], wr_ref[...],
                     preferred_element_type=jnp.float32)
           + bl_ref[...])
    if norm_relu:
        n = jnp.sqrt(jnp.sum(out * out, axis=1, keepdims=True))
        out = out / jnp.maximum(n, 1e-12)
        out = jnp.maximum(out, 0.0)
    o_ref[...] = out


def _tc_post(acc, cnt, xd, wl, bl, wr, norm_relu):
    return pl.pallas_call(
        functools.partial(_post_body, norm_relu),
        grid=(NMB,),
        in_specs=[
            pl.BlockSpec((2, MB, H), lambda m: (0, m, 0)),
            pl.BlockSpec((MB, 1), lambda m: (m, 0)),
            pl.BlockSpec((MB, D), lambda m: (m, 0)),
            pl.BlockSpec((2, H, D), lambda m: (0, 0, 0)),
            pl.BlockSpec((1, D), lambda m: (0, 0)),
            pl.BlockSpec((D, D), lambda m: (0, 0)),
        ],
        out_specs=pl.BlockSpec((MB, D), lambda m: (m, 0)),
        out_shape=jax.ShapeDtypeStruct((N, D), jnp.float32),
    )(acc, cnt, xd, wl, bl, wr)


def kernel(x_author, x_paper, edge_index, W_proj, b_proj, W_l, b_l, W_r):
    row2 = edge_index[0].reshape(NS, EPT)
    col2 = edge_index[1].reshape(NS, EPT)
    pad = EPTP - EPT
    row_gp = jnp.pad(row2, ((0, 0), (0, pad)))
    col_gp = jnp.pad(col2, ((0, 0), (0, pad)))
    row_g = jnp.stack([row_gp * 2, row_gp * 2 + 1]).reshape(
        2 * NS, NGRP, GSZ, CH)
    col_g = jnp.stack([col_gp * 2, col_gp * 2 + 1]).reshape(
        2 * NS, NGRP, GSZ, CH)
    row_s = jnp.pad(row2, ((0, 0), (0, pad)),
                    constant_values=N).reshape(NS, NCHUNK, CH)
    col_s = jnp.pad(col2, ((0, 0), (0, pad)),
                    constant_values=N).reshape(NS, NCHUNK, CH)
    zeros_h = jnp.zeros((CP_ROWS, H), jnp.float32)
    ones_h = jnp.ones((CH, H), jnp.float32)

    didx2 = jnp.concatenate([col_s, row_s], axis=0)
    counts = _sc_counts(didx2, zeros_h, ones_h)
    cnt_p = counts[0:N, 0:1]
    cnt_a = counts[N:2 * N, 0:1]

    def seg_both(ta_for_p, tp_for_a):
        op, oa = _sc_segsum(ta_for_p.reshape(2 * N, H),
                            tp_for_a.reshape(2 * N, H),
                            row_g, col_g, row_s, col_s, zeros_h)
        return op.reshape(2, N, H), oa.reshape(2, N, H)

    xa, xp = x_author, x_paper
    for i in range(4):
        ha = _tc_proj(xa, W_proj[i, 0], b_proj[i, 0].reshape(1, D))
        hp = _tc_proj(xp, W_proj[i, 1], b_proj[i, 1].reshape(1, D))
        sp, sa = seg_both(ha, hp)
        xp_new = _tc_post(sp, cnt_p, xp, W_l[i, 0].reshape(2, H, D),
                          b_l[i, 0].reshape(1, D), W_r[i, 0], True)
        xa_new = _tc_post(sa, cnt_a, xa, W_l[i, 1].reshape(2, H, D),
                          b_l[i, 1].reshape(1, D), W_r[i, 1], True)
        xp, xa = xp_new, xa_new

    sp, sa = seg_both(xa, xp)
    out_p = _tc_post(sp, cnt_p, xp, W_l[4, 0].reshape(2, H, D),
                     b_l[4, 0].reshape(1, D), W_r[4, 0], False)
    out_a = _tc_post(sa, cnt_a, xa, W_l[4, 1].reshape(2, H, D),
                     b_l[4, 1].reshape(1, D), W_r[4, 1], False)
    return (out_a, out_p)


# speedup vs baseline: 2.4085x; 1.0285x over previous
import functools

import jax
import jax.numpy as jnp
from jax import lax
from jax.experimental import pallas as pl
from jax.experimental.pallas import tpu as pltpu
from jax.experimental.pallas import tpu_sc as plsc

N = 10000
E = 160000
D = 256
H = 128
MB = 1000
NMB = N // MB
NS = 16
EPT = E // NS
CH = 112
NCHUNK = 90
EPTP = NCHUNK * CH
GSZ = 6
NGRP = NCHUNK // GSZ
NPAD = N + 8
CP_TILES = 10
CP_ROWS = N // CP_TILES


def _mesh():
    return plsc.VectorSubcoreMesh(
        core_axis_name="c", subcore_axis_name="s",
        num_cores=2, num_subcores=NS)


@functools.lru_cache(maxsize=None)
def _make_sc_segsum():
    return pl.kernel(
        _sc_segsum_body,
        out_type=(jax.ShapeDtypeStruct((2 * N, H), jnp.float32),
                  jax.ShapeDtypeStruct((2 * N, H), jnp.float32)),
        mesh=_mesh(),
        scratch_types=[
            pltpu.VMEM_SHARED((NPAD, H), jnp.float32),
            pltpu.VMEM((2, GSZ, CH), jnp.int32),
            [pltpu.VMEM((CH,), jnp.int32) for _ in range(GSZ)],
            [pltpu.VMEM((CH, H), jnp.float32) for _ in range(3)],
            pltpu.SemaphoreType.DMA,
            pltpu.SemaphoreType.DMA,
            pltpu.SemaphoreType.DMA,
            pltpu.SemaphoreType.DMA,
        ],
    )


def _sc_segsum(tp, ta, row_g, col_g, row_s, col_s, zeros):
    return _make_sc_segsum()(tp, ta, row_g, col_g, row_s, col_s, zeros)


def _one_direction(c, s, w, table_ref, sidx_ref, didx_ref, zeros_ref,
                   out_ref, acc, idx2b, drefs, bufs, gsem, ssem, isem, i2sem):
    @pl.when(s < CP_TILES)
    def _():
        pltpu.sync_copy(zeros_ref, acc.at[pl.ds(s * CP_ROWS, CP_ROWS)])

    plsc.subcore_barrier()

    def drain(sem, buf):
        pltpu.make_async_copy(table_ref.at[pl.ds(0, CH)], buf, sem).wait()

    def drain_idx(sem, ref):
        pltpu.make_async_copy(didx_ref.at[s, 0], ref, sem).wait()

    pltpu.sync_copy(sidx_ref.at[w, 0], idx2b.at[0])
    pltpu.async_copy(table_ref.at[idx2b.at[0, 0]], bufs[0], gsem)
    pltpu.async_copy(table_ref.at[idx2b.at[0, 1]], bufs[1], gsem)

    def group(g, carry):
        gp = g % 2

        @pl.when(g >= 1)
        def _():
            drain(ssem, bufs[2])

        @pl.when(g + 1 < NGRP)
        def _():
            pltpu.async_copy(sidx_ref.at[w, g + 1],
                             idx2b.at[(g + 1) % 2], i2sem)

        for jj in range(GSZ):
            pltpu.async_copy(didx_ref.at[s, g * GSZ + jj], drefs[jj], isem)
        for jj in range(GSZ):
            drain_idx(isem, drefs[jj])

        for jj in range(GSZ):
            j = g * GSZ + jj
            buf = bufs[jj % 3]
            fbuf = bufs[(jj + 2) % 3]

            if jj >= 1:
                drain(ssem, fbuf)

            if jj < GSZ - 2:
                @pl.when(j + 2 < NCHUNK)
                def _():
                    pltpu.async_copy(table_ref.at[idx2b.at[gp, jj + 2]],
                                     fbuf, gsem)
            else:
                @pl.when(j + 2 < NCHUNK)
                def _():
                    if jj == GSZ - 2:
                        pltpu.make_async_copy(sidx_ref.at[w, 0],
                                              idx2b.at[0], i2sem).wait()
                    pltpu.async_copy(
                        table_ref.at[idx2b.at[1 - gp, jj + 2 - GSZ]],
                        fbuf, gsem)

            drain(gsem, buf)
            pltpu.async_copy(buf, acc.at[drefs[jj]], ssem, add=True)
        return carry

    lax.fori_loop(0, NGRP, group, 0)
    drain(ssem, bufs[2])
    plsc.subcore_barrier()

    @pl.when(s < CP_TILES)
    def _():
        row0 = c * N + s * CP_ROWS
        pltpu.sync_copy(acc.at[pl.ds(s * CP_ROWS, CP_ROWS)],
                        out_ref.at[pl.ds(row0, CP_ROWS)])


def _sc_segsum_body(tp_ref, ta_ref, row_g_ref, col_g_ref, row_s_ref,
                    col_s_ref, zeros_ref, outp_ref, outa_ref,
                    acc, idx2b, drefs, bufs, gsem, ssem, isem, i2sem):
    c = lax.axis_index("c")
    s = lax.axis_index("s")
    w = c * NS + s
    _one_direction(c, s, w, tp_ref, row_g_ref, col_s_ref, zeros_ref,
                   outp_ref, acc, idx2b, drefs, bufs, gsem, ssem, isem, i2sem)
    plsc.subcore_barrier()
    _one_direction(c, s, w, ta_ref, col_g_ref, row_s_ref, zeros_ref,
                   outa_ref, acc, idx2b, drefs, bufs, gsem, ssem, isem, i2sem)


@functools.lru_cache(maxsize=None)
def _make_sc_counts():
    return pl.kernel(
        _sc_counts_body,
        out_type=jax.ShapeDtypeStruct((2 * N, H), jnp.float32),
        mesh=_mesh(),
        scratch_types=[
            pltpu.VMEM_SHARED((NPAD, H), jnp.float32),
            pltpu.VMEM((CH,), jnp.int32),
            pltpu.VMEM((CH,), jnp.int32),
            pltpu.VMEM((CH, H), jnp.float32),
            pltpu.SemaphoreType.DMA,
            pltpu.SemaphoreType.DMA,
        ],
    )


def _sc_counts(didx2, zeros, ones):
    return _make_sc_counts()(didx2, zeros, ones)


def _sc_counts_body(didx_ref, zeros_ref, ones_ref, out_ref, cacc, didx0,
                    didx1, ones_v, isem, ssem):
    c = lax.axis_index("c")
    s = lax.axis_index("s")
    ws = c * NS + s

    @pl.when(s < CP_TILES)
    def _():
        pltpu.sync_copy(zeros_ref, cacc.at[pl.ds(s * CP_ROWS, CP_ROWS)])

    pltpu.sync_copy(ones_ref, ones_v)
    plsc.subcore_barrier()
    refs = (didx0, didx1)
    pltpu.async_copy(didx_ref.at[ws, 0], didx0, isem)

    def drain_idx(ref):
        pltpu.make_async_copy(didx_ref.at[ws, 0], ref, isem).wait()

    def drain_sc(ref):
        pltpu.make_async_copy(zeros_ref.at[pl.ds(0, CH)], ones_v, ssem).wait()

    def body(g, carry):
        for par in range(2):
            i = g * 2 + par

            @pl.when(i >= 1)
            def _():
                drain_sc(None)

            @pl.when(i + 1 < NCHUNK)
            def _():
                pltpu.async_copy(didx_ref.at[ws, i + 1], refs[1 - par], isem)

            drain_idx(refs[par])
            pltpu.async_copy(ones_v, cacc.at[refs[par]], ssem, add=True)
        return carry

    lax.fori_loop(0, NCHUNK // 2, body, 0)
    drain_sc(None)
    plsc.subcore_barrier()

    @pl.when(s < CP_TILES)
    def _():
        row0 = c * N + s * CP_ROWS
        pltpu.sync_copy(cacc.at[pl.ds(s * CP_ROWS, CP_ROWS)],
                        out_ref.at[pl.ds(row0, CP_ROWS)])


def _proj_body(x_ref, w_ref, b_ref, o_ref):
    h = jnp.dot(x_ref[...], w_ref[...], preferred_element_type=jnp.float32)
    o_ref[...] = jnp.maximum(h + b_ref[...], 0.0)


def _tc_proj(x, w, b):
    return pl.pallas_call(
        _proj_body,
        grid=(NMB,),
        in_specs=[
            pl.BlockSpec((MB, D), lambda m: (m, 0)),
            pl.BlockSpec((D, D), lambda m: (0, 0)),
            pl.BlockSpec((1, D), lambda m: (0, 0)),
        ],
        out_specs=pl.BlockSpec((MB, D), lambda m: (m, 0)),
        out_shape=jax.ShapeDtypeStruct((N, D), jnp.float32),
    )(x, w, b)


def _post_body(norm_relu, acc_ref, cnt_ref, xd_ref, wl_ref, bl_ref, wr_ref,
               o_ref):
    denom = jnp.maximum(cnt_ref[...], 1.0)
    a0 = acc_ref[0] / denom
    a1 = acc_ref[1] / denom
    out = (jnp.dot(a0, wl_ref[0], preferred_element_type=jnp.float32)
           + jnp.dot(a1, wl_ref[1], preferred_element_type=jnp.float32)
           + jnp.dot(xd_ref[...], wr_ref[...],
                     preferred_element_type=jnp.float32)
           + bl_ref[...])
    if norm_relu:
        n = jnp.sqrt(jnp.sum(out * out, axis=1, keepdims=True))
        out = out / jnp.maximum(n, 1e-12)
        out = jnp.maximum(out, 0.0)
    o_ref[...] = out


def _tc_post(acc, cnt, xd, wl, bl, wr, norm_relu):
    return pl.pallas_call(
        functools.partial(_post_body, norm_relu),
        grid=(NMB,),
        in_specs=[
            pl.BlockSpec((2, MB, H), lambda m: (0, m, 0)),
            pl.BlockSpec((MB, 1), lambda m: (m, 0)),
            pl.BlockSpec((MB, D), lambda m: (m, 0)),
            pl.BlockSpec((2, H, D), lambda m: (0, 0, 0)),
            pl.BlockSpec((1, D), lambda m: (0, 0)),
            pl.BlockSpec((D, D), lambda m: (0, 0)),
        ],
        out_specs=pl.BlockSpec((MB, D), lambda m: (m, 0)),
        out_shape=jax.ShapeDtypeStruct((N, D), jnp.float32),
    )(acc, cnt, xd, wl, bl, wr)


def _postproj_body(acc_ref, cnt_ref, xd_ref, wl_ref, bl_ref, wr_ref,
                   wp_ref, bp_ref, ox_ref, oh_ref):
    denom = jnp.maximum(cnt_ref[...], 1.0)
    a0 = acc_ref[0] / denom
    a1 = acc_ref[1] / denom
    out = (jnp.dot(a0, wl_ref[0], preferred_element_type=jnp.float32)
           + jnp.dot(a1, wl_ref[1], preferred_element_type=jnp.float32)
           + jnp.dot(xd_ref[...], wr_ref[...],
                     preferred_element_type=jnp.float32)
           + bl_ref[...])
    n = jnp.sqrt(jnp.sum(out * out, axis=1, keepdims=True))
    out = out / jnp.maximum(n, 1e-12)
    out = jnp.maximum(out, 0.0)
    ox_ref[...] = out
    h = jnp.dot(out, wp_ref[...], preferred_element_type=jnp.float32)
    oh_ref[...] = jnp.maximum(h + bp_ref[...], 0.0)


def _tc_postproj(acc, cnt, xd, wl, bl, wr, wp, bp):
    return pl.pallas_call(
        _postproj_body,
        grid=(NMB,),
        in_specs=[
            pl.BlockSpec((2, MB, H), lambda m: (0, m, 0)),
            pl.BlockSpec((MB, 1), lambda m: (m, 0)),
            pl.BlockSpec((MB, D), lambda m: (m, 0)),
            pl.BlockSpec((2, H, D), lambda m: (0, 0, 0)),
            pl.BlockSpec((1, D), lambda m: (0, 0)),
            pl.BlockSpec((D, D), lambda m: (0, 0)),
            pl.BlockSpec((D, D), lambda m: (0, 0)),
            pl.BlockSpec((1, D), lambda m: (0, 0)),
        ],
        out_specs=[pl.BlockSpec((MB, D), lambda m: (m, 0)),
                   pl.BlockSpec((MB, D), lambda m: (m, 0))],
        out_shape=[jax.ShapeDtypeStruct((N, D), jnp.float32),
                   jax.ShapeDtypeStruct((N, D), jnp.float32)],
    )(acc, cnt, xd, wl, bl, wr, wp, bp)


def kernel(x_author, x_paper, edge_index, W_proj, b_proj, W_l, b_l, W_r):
    row2 = edge_index[0].reshape(NS, EPT)
    col2 = edge_index[1].reshape(NS, EPT)
    pad = EPTP - EPT
    row_gp = jnp.pad(row2, ((0, 0), (0, pad)))
    col_gp = jnp.pad(col2, ((0, 0), (0, pad)))
    row_g = jnp.stack([row_gp * 2, row_gp * 2 + 1]).reshape(
        2 * NS, NGRP, GSZ, CH)
    col_g = jnp.stack([col_gp * 2, col_gp * 2 + 1]).reshape(
        2 * NS, NGRP, GSZ, CH)
    row_s = jnp.pad(row2, ((0, 0), (0, pad)),
                    constant_values=N).reshape(NS, NCHUNK, CH)
    col_s = jnp.pad(col2, ((0, 0), (0, pad)),
                    constant_values=N).reshape(NS, NCHUNK, CH)
    zeros_h = jnp.zeros((CP_ROWS, H), jnp.float32)
    ones_h = jnp.ones((CH, H), jnp.float32)

    didx2 = jnp.concatenate([col_s, row_s], axis=0)
    counts = _sc_counts(didx2, zeros_h, ones_h)
    cnt_p = counts[0:N, 0:1]
    cnt_a = counts[N:2 * N, 0:1]

    def seg_both(ta_for_p, tp_for_a):
        op, oa = _sc_segsum(ta_for_p.reshape(2 * N, H),
                            tp_for_a.reshape(2 * N, H),
                            row_g, col_g, row_s, col_s, zeros_h)
        return op.reshape(2, N, H), oa.reshape(2, N, H)

    xa, xp = x_author, x_paper
    ha = _tc_proj(xa, W_proj[0, 0], b_proj[0, 0].reshape(1, D))
    hp = _tc_proj(xp, W_proj[0, 1], b_proj[0, 1].reshape(1, D))
    for i in range(4):
        sp, sa = seg_both(ha, hp)
        if i < 3:
            xp_new, hp = _tc_postproj(
                sp, cnt_p, xp, W_l[i, 0].reshape(2, H, D),
                b_l[i, 0].reshape(1, D), W_r[i, 0],
                W_proj[i + 1, 1], b_proj[i + 1, 1].reshape(1, D))
            xa_new, ha = _tc_postproj(
                sa, cnt_a, xa, W_l[i, 1].reshape(2, H, D),
                b_l[i, 1].reshape(1, D), W_r[i, 1],
                W_proj[i + 1, 0], b_proj[i + 1, 0].reshape(1, D))
        else:
            xp_new = _tc_post(sp, cnt_p, xp, W_l[i, 0].reshape(2, H, D),
                              b_l[i, 0].reshape(1, D), W_r[i, 0], True)
            xa_new = _tc_post(sa, cnt_a, xa, W_l[i, 1].reshape(2, H, D),
                              b_l[i, 1].reshape(1, D), W_r[i, 1], True)
        xp, xa = xp_new, xa_new

    sp, sa = seg_both(xa, xp)
    out_p = _tc_post(sp, cnt_p, xp, W_l[4, 0].reshape(2, H, D),
                     b_l[4, 0].reshape(1, D), W_r[4, 0], False)
    out_a = _tc_post(sa, cnt_a, xa, W_l[4, 1].reshape(2, H, D),
                     b_l[4, 1].reshape(1, D), W_r[4, 1], False)
    return (out_a, out_p)
